# baseline (speedup 1.0000x reference)
import functools

import jax
import jax.numpy as jnp
from jax import lax
from jax.experimental import pallas as pl
from jax.experimental.pallas import tpu as pltpu
from jax.experimental.pallas import tpu_sc as plsc

NUM_CORES = 2
NUM_SUBCORES = 16
NUM_WORKERS = NUM_CORES * NUM_SUBCORES
SUBLANE = 8
LANES = 16


@functools.lru_cache(maxsize=None)
def _make_lookup(V, D, Bt, H):
    Hp = (H + SUBLANE - 1) // SUBLANE * SUBLANE
    BLK = Bt // NUM_WORKERS
    n_grp = BLK // LANES
    assert Bt % NUM_WORKERS == 0 and BLK % LANES == 0 and H % 2 == 0
    mesh = plsc.VectorSubcoreMesh(core_axis_name="c", subcore_axis_name="s")

    @functools.partial(
        pl.kernel,
        mesh=mesh,
        out_type=jax.ShapeDtypeStruct((H, D, Bt), jnp.float32),
        scratch_types=[
            pltpu.VMEM((2, BLK), jnp.int32),
            pltpu.VMEM((2, BLK, D), jnp.float32),
            pltpu.VMEM((2, D, BLK), jnp.float32),
            pltpu.SemaphoreType.DMA((2,)),
            pltpu.SemaphoreType.DMA((2,)),
            pltpu.SemaphoreType.DMA((2,)),
        ],
        compiler_params=pltpu.CompilerParams(
            use_tc_tiling_on_sc=False, needs_layout_passes=False
        ),
    )
    def lookup(tab_hbm, idx_hbm, out_hbm, idx_v, rows_v, tr_v, sem_i, sem_g, sem_o):
        wid = lax.axis_index("s") * NUM_CORES + lax.axis_index("c")
        col0 = wid * BLK
        iota = lax.iota(jnp.int32, LANES)

        def idx_off(t):
            return t * Bt + col0

        def idx_start(t, b):
            pltpu.async_copy(
                idx_hbm.at[pl.ds(idx_off(t), BLK)], idx_v.at[b], sem_i.at[b]
            )

        def idx_wait(b):
            pltpu.make_async_copy(
                idx_hbm.at[pl.ds(0, BLK)], idx_v.at[b], sem_i.at[b]
            ).wait()

        def gather_start(b):
            pltpu.async_copy(tab_hbm.at[idx_v.at[b]], rows_v.at[b], sem_g.at[b])

        def gather_wait(b):
            pltpu.make_async_copy(
                tab_hbm.at[idx_v.at[b]], rows_v.at[b], sem_g.at[b]
            ).wait()

        def write_start(t, b):
            pltpu.async_copy(
                tr_v.at[b], out_hbm.at[t, :, pl.ds(col0, BLK)], sem_o.at[b]
            )

        def write_wait(b):
            pltpu.make_async_copy(
                tr_v.at[b], out_hbm.at[0, :, pl.ds(0, BLK)], sem_o.at[b]
            ).wait()

        def transpose(b):
            def g_body(g, carry):
                row_ids = g * LANES + iota
                for d in range(D):
                    col_ids = jnp.full((LANES,), d, jnp.int32)
                    vals = plsc.load_gather(rows_v.at[b], [row_ids, col_ids])
                    tr_v[b, d, pl.ds(g * LANES, LANES)] = vals
                return carry

            lax.fori_loop(0, n_grp, g_body, 0)

        for b in range(2):
            pltpu.sync_copy(idx_hbm.at[pl.ds(idx_off(b), BLK)], idx_v.at[b])
            gather_start(b)

        def super_body(s, carry):
            for j in range(2):
                t = s * 2 + j
                b = j

                @pl.when(s > 0)
                def _():
                    write_wait(b)

                gather_wait(b)

                @pl.when(s < (H // 2) - 1)
                def _():
                    idx_start(t + 2, b)

                transpose(b)
                write_start(t, b)

                @pl.when(s < (H // 2) - 1)
                def _():
                    idx_wait(b)
                    gather_start(b)

            return carry

        lax.fori_loop(0, H // 2, super_body, 0)

        for b in range(2):
            write_wait(b)

    return lookup


def kernel(input_seq, embedding_table):
    Bt, H = input_seq.shape
    V, D = embedding_table.shape
    Hp = (H + SUBLANE - 1) // SUBLANE * SUBLANE
    idxT = jnp.pad(input_seq.T.astype(jnp.int32), ((0, Hp - H), (0, 0)))
    outT = _make_lookup(V, D, Bt, H)(embedding_table, idxT.reshape(Hp * Bt))
    return outT.transpose(2, 0, 1)


# speedup vs baseline: 1.5123x; 1.5123x over previous
import functools

import jax
import jax.numpy as jnp
from jax import lax
from jax.experimental import pallas as pl
from jax.experimental.pallas import tpu as pltpu
from jax.experimental.pallas import tpu_sc as plsc

NUM_CORES = 2
NUM_SUBCORES = 16
NUM_WORKERS = NUM_CORES * NUM_SUBCORES
SUBLANE = 8
LANES = 16


@functools.lru_cache(maxsize=None)
def _make_lookup(V, D, Bt, H):
    Hp = (H + SUBLANE - 1) // SUBLANE * SUBLANE
    BLK = Bt // NUM_WORKERS
    n_rblk = BLK // LANES
    n_cblk = D // LANES
    assert Bt % NUM_WORKERS == 0 and BLK % LANES == 0 and D % LANES == 0
    assert H % 2 == 0
    mesh = plsc.VectorSubcoreMesh(core_axis_name="c", subcore_axis_name="s")

    @functools.partial(
        pl.kernel,
        mesh=mesh,
        out_type=jax.ShapeDtypeStruct((H, D, Bt), jnp.float32),
        scratch_types=[
            pltpu.VMEM((2, BLK), jnp.int32),
            pltpu.VMEM((2, BLK, D), jnp.float32),
            pltpu.VMEM((2, D, BLK), jnp.float32),
            pltpu.SemaphoreType.DMA((2,)),
            pltpu.SemaphoreType.DMA((2,)),
            pltpu.SemaphoreType.DMA((2,)),
        ],
        compiler_params=pltpu.CompilerParams(
            use_tc_tiling_on_sc=False, needs_layout_passes=False
        ),
    )
    def lookup(tab_hbm, idx_hbm, out_hbm, idx_v, rows_v, tr_v, sem_i, sem_g, sem_o):
        wid = lax.axis_index("s") * NUM_CORES + lax.axis_index("c")
        col0 = wid * BLK
        iota = lax.iota(jnp.int32, LANES)
        rots = [jnp.bitwise_and(iota + k, LANES - 1) for k in range(LANES)]

        def idx_off(t):
            return t * Bt + col0

        def idx_start(t, b):
            pltpu.async_copy(
                idx_hbm.at[pl.ds(idx_off(t), BLK)], idx_v.at[b], sem_i.at[b]
            )

        def idx_wait(b):
            pltpu.make_async_copy(
                idx_hbm.at[pl.ds(0, BLK)], idx_v.at[b], sem_i.at[b]
            ).wait()

        def gather_start(b):
            pltpu.async_copy(tab_hbm.at[idx_v.at[b]], rows_v.at[b], sem_g.at[b])

        def gather_wait(b):
            pltpu.make_async_copy(
                tab_hbm.at[idx_v.at[b]], rows_v.at[b], sem_g.at[b]
            ).wait()

        def write_start(t, b):
            pltpu.async_copy(
                tr_v.at[b], out_hbm.at[t, :, pl.ds(col0, BLK)], sem_o.at[b]
            )

        def write_wait(b):
            pltpu.make_async_copy(
                tr_v.at[b], out_hbm.at[0, :, pl.ds(0, BLK)], sem_o.at[b]
            ).wait()

        def transpose(b):
            def r_body(r0, carry):
                row_ids = r0 * LANES + iota
                for c0 in range(n_cblk):
                    for k in range(LANES):
                        col_ids = c0 * LANES + rots[k]
                        vals = plsc.load_gather(
                            rows_v.at[b], [row_ids, col_ids]
                        )
                        plsc.store_scatter(
                            tr_v.at[b], [col_ids, row_ids], vals
                        )
                return carry

            lax.fori_loop(0, n_rblk, r_body, 0)

        for b in range(2):
            pltpu.sync_copy(idx_hbm.at[pl.ds(idx_off(b), BLK)], idx_v.at[b])
            gather_start(b)

        def super_body(s, carry):
            for j in range(2):
                t = s * 2 + j
                b = j

                @pl.when(s > 0)
                def _():
                    write_wait(b)

                gather_wait(b)

                @pl.when(s < (H // 2) - 1)
                def _():
                    idx_start(t + 2, b)

                transpose(b)
                write_start(t, b)

                @pl.when(s < (H // 2) - 1)
                def _():
                    idx_wait(b)
                    gather_start(b)

            return carry

        lax.fori_loop(0, H // 2, super_body, 0)

        for b in range(2):
            write_wait(b)

    return lookup


def kernel(input_seq, embedding_table):
    Bt, H = input_seq.shape
    V, D = embedding_table.shape
    Hp = (H + SUBLANE - 1) // SUBLANE * SUBLANE
    idxT = jnp.pad(input_seq.T.astype(jnp.int32), ((0, Hp - H), (0, 0)))
    outT = _make_lookup(V, D, Bt, H)(embedding_table, idxT.reshape(Hp * Bt))
    return outT.transpose(2, 0, 1)
